# Initial kernel scaffold; baseline (speedup 1.0000x reference)
import functools
import math

import jax
import jax.numpy as jnp
from jax.experimental import pallas as pl
from jax.experimental.pallas import tpu as pltpu

_MARGIN = 0.2
_P_TARGET = 0.1
_BLOCK_ROWS = 1024


def _loss_kernel(dist_ref, labels_ref, out_ref, acc_ref, *, n_steps):
    i = pl.program_id(0)
    blk = dist_ref[...]
    labels = labels_ref[...]
    col = jax.lax.broadcasted_iota(jnp.int32, blk.shape, 1)
    is_pos = col == labels
    pos_min_blk = jnp.min(jnp.where(is_pos, blk, jnp.inf))
    neg_max_blk = jnp.max(jnp.where(is_pos, -jnp.inf, blk))

    @pl.when(i == 0)
    def _init():
        acc_ref[0] = pos_min_blk
        acc_ref[1] = neg_max_blk

    @pl.when(i > 0)
    def _accum():
        acc_ref[0] = jnp.minimum(acc_ref[0], pos_min_blk)
        acc_ref[1] = jnp.maximum(acc_ref[1], neg_max_blk)

    @pl.when(i == n_steps - 1)
    def _finish():
        pos_theta = jnp.arccos(acc_ref[0])
        neg_theta = jnp.arccos(acc_ref[1])
        loss = _P_TARGET * jnp.maximum(pos_theta, _MARGIN) + (
            _P_TARGET - 1.0
        ) * jnp.minimum(neg_theta, 0.5 * math.pi - _MARGIN)
        out_ref[0, 0] = loss


@jax.jit
def kernel(dist, labels):
    b, c = dist.shape
    n_steps = b // _BLOCK_ROWS
    labels2 = labels.reshape(b, 1)
    out = pl.pallas_call(
        functools.partial(_loss_kernel, n_steps=n_steps),
        grid=(n_steps,),
        in_specs=[
            pl.BlockSpec((_BLOCK_ROWS, c), lambda i: (i, 0)),
            pl.BlockSpec((_BLOCK_ROWS, 1), lambda i: (i, 0)),
        ],
        out_specs=pl.BlockSpec((1, 1), lambda i: (0, 0),
                               memory_space=pltpu.SMEM),
        out_shape=jax.ShapeDtypeStruct((1, 1), jnp.float32),
        scratch_shapes=[pltpu.SMEM((2,), jnp.float32)],
    )(dist, labels2)
    return out[0, 0]


# speedup vs baseline: 651.7443x; 651.7443x over previous
import functools
import math

import jax
import jax.numpy as jnp
from jax.experimental import pallas as pl
from jax.experimental.pallas import tpu as pltpu

_MARGIN = 0.2
_P_TARGET = 0.1
_BLOCK_ROWS = 1024


def _loss_kernel(dist_ref, labels_ref, out_ref, acc_ref, *, n_steps):
    i = pl.program_id(0)
    blk = dist_ref[...]
    labels = labels_ref[...]
    col = jax.lax.broadcasted_iota(jnp.int32, blk.shape, 1)
    is_pos = col == labels
    pos_min_blk = jnp.min(jnp.where(is_pos, blk, jnp.inf))
    neg_max_blk = jnp.max(jnp.where(is_pos, -jnp.inf, blk))

    @pl.when(i == 0)
    def _init():
        acc_ref[0] = pos_min_blk
        acc_ref[1] = neg_max_blk

    @pl.when(i > 0)
    def _accum():
        acc_ref[0] = jnp.minimum(acc_ref[0], pos_min_blk)
        acc_ref[1] = jnp.maximum(acc_ref[1], neg_max_blk)

    @pl.when(i == n_steps - 1)
    def _finish():
        out_ref[0] = acc_ref[0]
        out_ref[1] = acc_ref[1]


@jax.jit
def kernel(dist, labels):
    b, c = dist.shape
    n_steps = b // _BLOCK_ROWS
    labels2 = labels.reshape(b, 1)
    out = pl.pallas_call(
        functools.partial(_loss_kernel, n_steps=n_steps),
        grid=(n_steps,),
        in_specs=[
            pl.BlockSpec((_BLOCK_ROWS, c), lambda i: (i, 0)),
            pl.BlockSpec((_BLOCK_ROWS, 1), lambda i: (i, 0)),
        ],
        out_specs=pl.BlockSpec((2,), lambda i: (0,),
                               memory_space=pltpu.SMEM),
        out_shape=jax.ShapeDtypeStruct((2,), jnp.float32),
        scratch_shapes=[pltpu.SMEM((2,), jnp.float32)],
    )(dist, labels2)
    pos_min, neg_max = out[0], out[1]
    pos_theta = jnp.arccos(pos_min)
    neg_theta = jnp.arccos(neg_max)
    return _P_TARGET * jnp.maximum(pos_theta, _MARGIN) + (
        _P_TARGET - 1.0
    ) * jnp.minimum(neg_theta, 0.5 * math.pi - _MARGIN)


# speedup vs baseline: 675.6061x; 1.0366x over previous
import functools
import math

import jax
import jax.numpy as jnp
from jax.experimental import pallas as pl
from jax.experimental.pallas import tpu as pltpu

_MARGIN = 0.2
_P_TARGET = 0.1
_BLOCK_ROWS = 2048


def _loss_kernel(dist_ref, labels_ref, out_ref, acc_ref, *, n_steps):
    i = pl.program_id(0)
    blk = dist_ref[...]
    labels = labels_ref[...]
    col = jax.lax.broadcasted_iota(jnp.int32, blk.shape, 1)
    is_pos = col == labels
    pos_min_blk = jnp.min(jnp.where(is_pos, blk, jnp.inf))
    neg_max_blk = jnp.max(jnp.where(is_pos, -jnp.inf, blk))

    @pl.when(i == 0)
    def _init():
        acc_ref[0] = pos_min_blk
        acc_ref[1] = neg_max_blk

    @pl.when(i > 0)
    def _accum():
        acc_ref[0] = jnp.minimum(acc_ref[0], pos_min_blk)
        acc_ref[1] = jnp.maximum(acc_ref[1], neg_max_blk)

    @pl.when(i == n_steps - 1)
    def _finish():
        out_ref[0] = acc_ref[0]
        out_ref[1] = acc_ref[1]


@jax.jit
def kernel(dist, labels):
    b, c = dist.shape
    n_steps = b // _BLOCK_ROWS
    labels2 = labels.reshape(b, 1)
    out = pl.pallas_call(
        functools.partial(_loss_kernel, n_steps=n_steps),
        grid=(n_steps,),
        in_specs=[
            pl.BlockSpec((_BLOCK_ROWS, c), lambda i: (i, 0)),
            pl.BlockSpec((_BLOCK_ROWS, 1), lambda i: (i, 0)),
        ],
        out_specs=pl.BlockSpec((2,), lambda i: (0,),
                               memory_space=pltpu.SMEM),
        out_shape=jax.ShapeDtypeStruct((2,), jnp.float32),
        scratch_shapes=[pltpu.SMEM((2,), jnp.float32)],
    )(dist, labels2)
    pos_min, neg_max = out[0], out[1]
    pos_theta = jnp.arccos(pos_min)
    neg_theta = jnp.arccos(neg_max)
    return _P_TARGET * jnp.maximum(pos_theta, _MARGIN) + (
        _P_TARGET - 1.0
    ) * jnp.minimum(neg_theta, 0.5 * math.pi - _MARGIN)


# speedup vs baseline: 675.6708x; 1.0001x over previous
import functools
import math

import jax
import jax.numpy as jnp
from jax.experimental import pallas as pl
from jax.experimental.pallas import tpu as pltpu

_MARGIN = 0.2
_P_TARGET = 0.1
_BLOCK_ROWS = 4096


def _loss_kernel(dist_ref, labels_ref, out_ref, acc_ref, *, n_steps):
    i = pl.program_id(0)
    blk = dist_ref[...]
    labels = labels_ref[...]
    col = jax.lax.broadcasted_iota(jnp.int32, blk.shape, 1)
    is_pos = col == labels
    pos_min_blk = jnp.min(jnp.where(is_pos, blk, jnp.inf))
    neg_max_blk = jnp.max(jnp.where(is_pos, -jnp.inf, blk))

    @pl.when(i == 0)
    def _init():
        acc_ref[0] = pos_min_blk
        acc_ref[1] = neg_max_blk

    @pl.when(i > 0)
    def _accum():
        acc_ref[0] = jnp.minimum(acc_ref[0], pos_min_blk)
        acc_ref[1] = jnp.maximum(acc_ref[1], neg_max_blk)

    @pl.when(i == n_steps - 1)
    def _finish():
        out_ref[0] = acc_ref[0]
        out_ref[1] = acc_ref[1]


@jax.jit
def kernel(dist, labels):
    b, c = dist.shape
    n_steps = b // _BLOCK_ROWS
    labels2 = labels.reshape(b, 1)
    out = pl.pallas_call(
        functools.partial(_loss_kernel, n_steps=n_steps),
        grid=(n_steps,),
        in_specs=[
            pl.BlockSpec((_BLOCK_ROWS, c), lambda i: (i, 0)),
            pl.BlockSpec((_BLOCK_ROWS, 1), lambda i: (i, 0)),
        ],
        out_specs=pl.BlockSpec((2,), lambda i: (0,),
                               memory_space=pltpu.SMEM),
        out_shape=jax.ShapeDtypeStruct((2,), jnp.float32),
        scratch_shapes=[pltpu.SMEM((2,), jnp.float32)],
    )(dist, labels2)
    pos_min, neg_max = out[0], out[1]
    pos_theta = jnp.arccos(pos_min)
    neg_theta = jnp.arccos(neg_max)
    return _P_TARGET * jnp.maximum(pos_theta, _MARGIN) + (
        _P_TARGET - 1.0
    ) * jnp.minimum(neg_theta, 0.5 * math.pi - _MARGIN)


# speedup vs baseline: 693.4786x; 1.0264x over previous
import functools
import math

import jax
import jax.numpy as jnp
from jax.experimental import pallas as pl
from jax.experimental.pallas import tpu as pltpu

_MARGIN = 0.2
_P_TARGET = 0.1
_BLOCK_ROWS = 1024


def _loss_kernel(dist_a_ref, dist_b_ref, labels_a_ref, labels_b_ref,
                 out_ref, acc_ref, *, n_steps):
    i = pl.program_id(0)

    def masked_stats(blk, labels):
        col = jax.lax.broadcasted_iota(jnp.int32, blk.shape, 1)
        is_pos = col == labels
        pos = jnp.min(jnp.where(is_pos, blk, jnp.inf))
        neg = jnp.max(jnp.where(is_pos, -jnp.inf, blk))
        return pos, neg

    pos_a, neg_a = masked_stats(dist_a_ref[...], labels_a_ref[...])
    pos_b, neg_b = masked_stats(dist_b_ref[...], labels_b_ref[...])
    pos_min_blk = jnp.minimum(pos_a, pos_b)
    neg_max_blk = jnp.maximum(neg_a, neg_b)

    @pl.when(i == 0)
    def _init():
        acc_ref[0] = pos_min_blk
        acc_ref[1] = neg_max_blk

    @pl.when(i > 0)
    def _accum():
        acc_ref[0] = jnp.minimum(acc_ref[0], pos_min_blk)
        acc_ref[1] = jnp.maximum(acc_ref[1], neg_max_blk)

    @pl.when(i == n_steps - 1)
    def _finish():
        out_ref[0] = acc_ref[0]
        out_ref[1] = acc_ref[1]


@jax.jit
def kernel(dist, labels):
    b, c = dist.shape
    half = b // 2
    n_steps = half // _BLOCK_ROWS
    labels2 = labels.reshape(b, 1)
    out = pl.pallas_call(
        functools.partial(_loss_kernel, n_steps=n_steps),
        grid=(n_steps,),
        in_specs=[
            pl.BlockSpec((_BLOCK_ROWS, c), lambda i: (i, 0)),
            pl.BlockSpec((_BLOCK_ROWS, c),
                         lambda i: (i + n_steps, 0)),
            pl.BlockSpec((_BLOCK_ROWS, 1), lambda i: (i, 0)),
            pl.BlockSpec((_BLOCK_ROWS, 1),
                         lambda i: (i + n_steps, 0)),
        ],
        out_specs=pl.BlockSpec((2,), lambda i: (0,),
                               memory_space=pltpu.SMEM),
        out_shape=jax.ShapeDtypeStruct((2,), jnp.float32),
        scratch_shapes=[pltpu.SMEM((2,), jnp.float32)],
    )(dist, dist, labels2, labels2)
    pos_min, neg_max = out[0], out[1]
    pos_theta = jnp.arccos(pos_min)
    neg_theta = jnp.arccos(neg_max)
    return _P_TARGET * jnp.maximum(pos_theta, _MARGIN) + (
        _P_TARGET - 1.0
    ) * jnp.minimum(neg_theta, 0.5 * math.pi - _MARGIN)
